# Initial kernel scaffold; baseline (speedup 1.0000x reference)
import jax
import jax.numpy as jnp
from jax.experimental import pallas as pl


def kernel(z, pos, batch, emb, Wp, W1, Wv, Wo1, Wo2):
    raise NotImplementedError("write your pallas kernel here")


# speedup vs baseline: 3.7692x; 3.7692x over previous
import functools

import jax
import jax.numpy as jnp
from jax.experimental import pallas as pl
from jax.experimental.pallas import tpu as pltpu

_N = 16384
_H = 256
_NG = 256
_ZMAX = 100
_ZPAD = 128
_B = 1024
_NB = _N // _B


def _head_body(z_ref, pos_ref, batch_ref, emb_ref, Wp_ref, Wv_ref, Wo1_ref,
               Wo2_ref, W1_ref, out_ref, embW1_ref):
    i = pl.program_id(0)

    @pl.when(i == 0)
    def _():
        embW1_ref[...] = jnp.dot(emb_ref[...], W1_ref[...],
                                 preferred_element_type=jnp.float32)
        out_ref[...] = jnp.zeros_like(out_ref)

    z = z_ref[...]
    oh = (z == jax.lax.broadcasted_iota(jnp.int32, (_B, _ZPAD), 1)
          ).astype(jnp.float32)
    posb = pos_ref[...]
    x = jnp.dot(oh, embW1_ref[...], preferred_element_type=jnp.float32)
    x = x + jnp.dot(posb, Wp_ref[...], preferred_element_type=jnp.float32)
    x = x * jax.nn.sigmoid(x)
    u = jnp.dot(x, Wv_ref[...], preferred_element_type=jnp.float32)
    pos2 = jnp.sum(posb * posb, axis=1, keepdims=True)
    g = x + pos2 * (u * u)
    s = jnp.dot(g, Wo1_ref[...], preferred_element_type=jnp.float32)
    s = s * jax.nn.sigmoid(s)
    pa = jnp.dot(s, Wo2_ref[...], preferred_element_type=jnp.float32)

    bat = batch_ref[...].reshape(1, _B)
    seg = (jax.lax.broadcasted_iota(jnp.int32, (_NG, _B), 0) == bat
           ).astype(jnp.float32)
    out_ref[...] += jnp.dot(seg, pa, preferred_element_type=jnp.float32)


@functools.partial(jax.jit, static_argnames=())
def _head(z2, pos_pad, batch3, emb_pad, Wp_pad, Wv, Wo1, Wo2, W1):
    return pl.pallas_call(
        _head_body,
        grid=(_NB,),
        in_specs=[
            pl.BlockSpec((_B, 1), lambda i: (i, 0)),
            pl.BlockSpec((_B, 8), lambda i: (i, 0)),
            pl.BlockSpec((1, 1, _B), lambda i: (i, 0, 0)),
            pl.BlockSpec((_ZPAD, _H), lambda i: (0, 0)),
            pl.BlockSpec((8, _H), lambda i: (0, 0)),
            pl.BlockSpec((_H, _H), lambda i: (0, 0)),
            pl.BlockSpec((_H, _H // 2), lambda i: (0, 0)),
            pl.BlockSpec((_H // 2, 1), lambda i: (0, 0)),
            pl.BlockSpec((_H, _H), lambda i: (0, 0)),
        ],
        out_specs=pl.BlockSpec((_NG, 1), lambda i: (0, 0)),
        out_shape=jax.ShapeDtypeStruct((_NG, 1), jnp.float32),
        scratch_shapes=[pltpu.VMEM((_ZPAD, _H), jnp.float32)],
    )(z2, pos_pad, batch3, emb_pad, Wp_pad, Wv, Wo1, Wo2, W1)


def kernel(z, pos, batch, emb, Wp, W1, Wv, Wo1, Wo2):
    z2 = z.astype(jnp.int32).reshape(_N, 1)
    pos_pad = jnp.pad(pos, ((0, 0), (0, 5)))
    batch3 = batch.astype(jnp.int32).reshape(_NB, 1, _B)
    emb_pad = jnp.pad(emb, ((0, _ZPAD - _ZMAX), (0, 0)))
    Wp_pad = jnp.pad(Wp, ((0, 5), (0, 0)))
    return _head(z2, pos_pad, batch3, emb_pad, Wp_pad, Wv, Wo1, Wo2, W1)
